# Initial kernel scaffold; baseline (speedup 1.0000x reference)
import jax
import jax.numpy as jnp
from jax.experimental import pallas as pl


def kernel(input_ids, embeddings):
    raise NotImplementedError("write your pallas kernel here")


# speedup vs baseline: 3.3456x; 3.3456x over previous
import functools

import jax
import jax.numpy as jnp
from jax import lax
from jax.experimental import pallas as pl
from jax.experimental.pallas import tpu as pltpu
from jax.experimental.pallas import tpu_sc as plsc

_BATCH, _HIST, _DIM = 4096, 50, 128
_NW = 32
_PER_W = (_BATCH * _HIST) // _NW
_CH = 128
_NCH = _PER_W // _CH
_NBUF = 5
_NROUND = _NCH // _NBUF


def _sc_gather(ids, table):
    mesh = plsc.VectorSubcoreMesh(core_axis_name="c", subcore_axis_name="s")
    scratch = [pltpu.VMEM((_NCH, _CH), jnp.int32)]
    scratch += [pltpu.VMEM((_CH, _DIM), jnp.float32) for _ in range(_NBUF)]
    scratch += [pltpu.SemaphoreType.DMA for _ in range(_NBUF)]

    @functools.partial(
        pl.kernel,
        out_type=jax.ShapeDtypeStruct((_BATCH * _HIST, _DIM), jnp.float32),
        mesh=mesh,
        scratch_types=scratch,
    )
    def k(ids_hbm, table_hbm, out_hbm, idx_v, *rest):
        bufs = rest[:_NBUF]
        sems = rest[_NBUF:]
        wid = lax.axis_index("s") * 2 + lax.axis_index("c")
        base = wid * _PER_W
        pltpu.sync_copy(ids_hbm.at[wid], idx_v)
        for b in range(_NBUF):
            pltpu.async_copy(table_hbm.at[idx_v.at[b]], bufs[b], sems[b])

        def round_body(o, carry):
            for b in range(_NBUF):
                g = o * _NBUF + b
                pltpu.make_async_copy(
                    table_hbm.at[idx_v.at[g]], bufs[b], sems[b]).wait()
                pltpu.sync_copy(bufs[b], out_hbm.at[pl.ds(base + g * _CH, _CH)])
                pltpu.async_copy(
                    table_hbm.at[idx_v.at[g + _NBUF]], bufs[b], sems[b])
            return carry

        lax.fori_loop(0, _NROUND - 1, round_body, 0)

        o = _NROUND - 1
        for b in range(_NBUF):
            g = o * _NBUF + b
            pltpu.make_async_copy(
                table_hbm.at[idx_v.at[g]], bufs[b], sems[b]).wait()
            pltpu.sync_copy(bufs[b], out_hbm.at[pl.ds(base + g * _CH, _CH)])

    return k(ids, table)


def kernel(input_ids, embeddings):
    ids = input_ids.reshape(_NW, _NCH, _CH).astype(jnp.int32)
    out = _sc_gather(ids, embeddings)
    return out.reshape(_BATCH, _HIST, _DIM)


# speedup vs baseline: 5.9633x; 1.7825x over previous
import functools

import jax
import jax.numpy as jnp
from jax import lax
from jax.experimental import pallas as pl
from jax.experimental.pallas import tpu as pltpu
from jax.experimental.pallas import tpu_sc as plsc

_BATCH, _HIST, _DIM = 4096, 50, 128
_HPAD = 56
_NW = 32
_PER_W = _BATCH // _NW
_NBUF = 8
_NROUND = _PER_W // _NBUF


def _sc_gather(ids, table):
    mesh = plsc.VectorSubcoreMesh(core_axis_name="c", subcore_axis_name="s")
    scratch = [pltpu.VMEM((_PER_W, _HPAD), jnp.int32)]
    scratch += [pltpu.VMEM((_HIST, _DIM), jnp.float32) for _ in range(_NBUF)]
    scratch += [pltpu.SemaphoreType.DMA for _ in range(_NBUF)]

    @functools.partial(
        pl.kernel,
        out_type=jax.ShapeDtypeStruct((_BATCH, _HIST, _DIM), jnp.float32),
        mesh=mesh,
        scratch_types=scratch,
    )
    def k(ids_hbm, table_hbm, out_hbm, idx_v, *rest):
        bufs = rest[:_NBUF]
        sems = rest[_NBUF:]
        wid = lax.axis_index("s") * 2 + lax.axis_index("c")
        base = wid * _PER_W
        pltpu.sync_copy(ids_hbm.at[wid], idx_v)
        for b in range(_NBUF):
            pltpu.async_copy(
                table_hbm.at[idx_v.at[b, pl.ds(0, _HIST)]], bufs[b], sems[b])

        def round_body(o, carry):
            for b in range(_NBUF):
                g = o * _NBUF + b
                pltpu.make_async_copy(
                    table_hbm.at[idx_v.at[g, pl.ds(0, _HIST)]],
                    bufs[b], sems[b]).wait()
                pltpu.sync_copy(bufs[b], out_hbm.at[base + g])
                pltpu.async_copy(
                    table_hbm.at[idx_v.at[g + _NBUF, pl.ds(0, _HIST)]],
                    bufs[b], sems[b])
            return carry

        lax.fori_loop(0, _NROUND - 1, round_body, 0)

        o = _NROUND - 1
        for b in range(_NBUF):
            g = o * _NBUF + b
            pltpu.make_async_copy(
                table_hbm.at[idx_v.at[g, pl.ds(0, _HIST)]],
                bufs[b], sems[b]).wait()
            pltpu.sync_copy(bufs[b], out_hbm.at[base + g])

    return k(ids, table)


def kernel(input_ids, embeddings):
    ids = jnp.pad(input_ids.astype(jnp.int32), ((0, 0), (0, _HPAD - _HIST)))
    ids = ids.reshape(_NW, _PER_W, _HPAD)
    return _sc_gather(ids, embeddings)


# speedup vs baseline: 10.7339x; 1.8000x over previous
import functools

import jax
import jax.numpy as jnp
from jax import lax
from jax.experimental import pallas as pl
from jax.experimental.pallas import tpu as pltpu
from jax.experimental.pallas import tpu_sc as plsc

_BATCH, _HIST, _DIM = 4096, 50, 128
_NW = 32
_PER_W = _BATCH // _NW
_NBUF = 5
_NROUND = _HIST // _NBUF


def _sc_gather(ids_t, table):
    mesh = plsc.VectorSubcoreMesh(core_axis_name="c", subcore_axis_name="s")
    scratch = [pltpu.VMEM((_HIST, _PER_W), jnp.int32)]
    scratch += [pltpu.VMEM((_PER_W, _DIM), jnp.float32) for _ in range(_NBUF)]
    scratch += [pltpu.SemaphoreType.DMA for _ in range(_NBUF)]

    @functools.partial(
        pl.kernel,
        out_type=jax.ShapeDtypeStruct((_HIST, _BATCH, _DIM), jnp.float32),
        mesh=mesh,
        scratch_types=scratch,
    )
    def k(ids_hbm, table_hbm, out_hbm, idx_v, *rest):
        bufs = rest[:_NBUF]
        sems = rest[_NBUF:]
        wid = lax.axis_index("s") * 2 + lax.axis_index("c")
        b0 = wid * _PER_W
        pltpu.sync_copy(ids_hbm.at[:, pl.ds(b0, _PER_W)], idx_v)
        for b in range(_NBUF):
            pltpu.async_copy(table_hbm.at[idx_v.at[b]], bufs[b], sems[b])

        def round_body(o, carry):
            for b in range(_NBUF):
                h = o * _NBUF + b
                pltpu.make_async_copy(
                    table_hbm.at[idx_v.at[h]], bufs[b], sems[b]).wait()
                pltpu.sync_copy(bufs[b], out_hbm.at[h, pl.ds(b0, _PER_W)])
                pltpu.async_copy(
                    table_hbm.at[idx_v.at[h + _NBUF]], bufs[b], sems[b])
            return carry

        lax.fori_loop(0, _NROUND - 1, round_body, 0)

        o = _NROUND - 1
        for b in range(_NBUF):
            h = o * _NBUF + b
            pltpu.make_async_copy(
                table_hbm.at[idx_v.at[h]], bufs[b], sems[b]).wait()
            pltpu.sync_copy(bufs[b], out_hbm.at[h, pl.ds(b0, _PER_W)])

    return k(ids_t, table)


def kernel(input_ids, embeddings):
    ids_t = input_ids.T.astype(jnp.int32)
    out = _sc_gather(ids_t, embeddings)
    return out.transpose(1, 0, 2)
